# baseline (speedup 1.0000x reference)
import jax
import jax.numpy as jnp
from jax import lax
from jax.experimental import pallas as pl
from jax.experimental.pallas import tpu as pltpu
from jax.experimental.pallas import tpu_sc as plsc

B, N, D = 16, 4096, 256
NC, NS = 2, 16
BPC = B // NC
NV = N // 16
HALF = N // 2
CHUNK = 64
NBUF = 4
NGRP = HALF // (NBUF * CHUNK)
INF_BITS = 0x7F800000


def _body(x_hbm, mag_hbm, mask_hbm, out_hbm,
          fbuf, mbuf, kb0, ib0, kb1, ib1, hist, h2, obuf, dbuf, rnk, tot,
          idxbuf, rows0, rows1, rows2, rows3, perm_sh,
          sem0, sem1, sem2, sem3):
    c = lax.axis_index("c")
    s = lax.axis_index("s")
    lane = lax.iota(jnp.int32, 16)
    rows = [rows0, rows1, rows2, rows3]
    sems = [sem0, sem1, sem2, sem3]

    @pl.when(s < BPC)
    def _sort():
        b = c * BPC + s
        pltpu.sync_copy(mag_hbm.at[b], fbuf)
        pltpu.sync_copy(mask_hbm.at[b], mbuf)

        zeros = jnp.zeros((16,), jnp.int32)
        ones = jnp.ones((16,), jnp.int32)

        @plsc.parallel_loop(0, NV, 1, unroll=4)
        def _init(t):
            ki = jnp.bitwise_and(fbuf[pl.ds(t * 16, 16)], 0x7FFFFFFF)
            m = mbuf[pl.ds(t * 16, 16)]
            e = t * 16 + lane
            a = jnp.bitwise_and(e, 255)
            addr = a * 16 + jnp.bitwise_and(jnp.right_shift(e, 8) + a, 15)
            plsc.store_scatter(kb0, [addr], jnp.where(m != 0, ki, INF_BITS))
            ib0[pl.ds(t * 16, 16)] = jnp.bitwise_and(lane - t, 15) * 256 + t
            hist[pl.ds(t * 16, 16)] = zeros
            h2[pl.ds(t * 16, 16)] = zeros

        bufs = [(kb0, ib0), (kb1, ib1)]
        for p in range(4):
            kin, iin = bufs[p % 2]
            kout, iout = bufs[(p + 1) % 2]
            shift = 8 * p

            @plsc.parallel_loop(0, NV, 1, unroll=4)
            def _dig(t):
                k = kin[pl.ds(t * 16, 16)]
                lv = jnp.bitwise_and(lane - t, 15)
                dbuf[pl.ds(t * 16, 16)] = jnp.bitwise_and(
                    jnp.right_shift(k, shift), 255) * 16 + lv
                if p > 0:
                    hist[pl.ds(t * 16, 16)] = zeros
                    h2[pl.ds(t * 16, 16)] = zeros

            def hist_body(i, carry):
                for m, hm in ((0, hist), (1, h2)):
                    t = m * (NV // 2) + i
                    hidx = dbuf[pl.ds(t * 16, 16)]
                    r = plsc.load_gather(hm, [hidx])
                    rnk[pl.ds(t * 16, 16)] = r
                    plsc.addupdate_scatter(hm, [hidx], ones)
                return carry
            lax.fori_loop(0, NV // 2, hist_body, jnp.int32(0), unroll=2)

            @plsc.parallel_loop(0, 256, 1, unroll=4)
            def _scan_a(dd):
                h = hist[pl.ds(dd * 16, 16)] + h2[pl.ds(dd * 16, 16)]
                obuf[pl.ds(dd * 16, 16)] = jnp.cumsum(h) - h
                tot[dd] = jnp.sum(h)

            def scan_b(dd, carry):
                t = tot[dd]
                tot[dd] = carry
                return carry + t
            lax.fori_loop(0, 256, scan_b, jnp.int32(0), unroll=4)

            @plsc.parallel_loop(0, 256, 1, unroll=4)
            def _scan_c(dd):
                obuf[pl.ds(dd * 16, 16)] = obuf[pl.ds(dd * 16, 16)] + tot[dd]

            last = p == 3

            @plsc.parallel_loop(0, NV // 2, 1, unroll=2)
            def _perm(i):
                for m in (0, 1):
                    t = m * (NV // 2) + i
                    k = kin[pl.ds(t * 16, 16)]
                    v = iin[pl.ds(t * 16, 16)]
                    hidx = dbuf[pl.ds(t * 16, 16)]
                    pos = (plsc.load_gather(obuf, [hidx])
                           + rnk[pl.ds(t * 16, 16)])
                    if m == 1:
                        pos = pos + plsc.load_gather(hist, [hidx])
                    if not last:
                        a2 = jnp.bitwise_and(pos, 255)
                        pos = a2 * 16 + jnp.bitwise_and(
                            jnp.right_shift(pos, 8) + a2, 15)
                    plsc.store_scatter(kout, [pos], k)
                    plsc.store_scatter(iout, [pos], v)

        base_row = b * N

        @plsc.parallel_loop(0, NV, 1, unroll=4)
        def _add(t):
            ib0[pl.ds(t * 16, 16)] = ib0[pl.ds(t * 16, 16)] + base_row
        pltpu.sync_copy(ib0, perm_sh.at[s])

    plsc.subcore_barrier()

    j = s // 2
    b = c * BPC + j
    out_base = b * N + (s % 2) * HALF
    pltpu.sync_copy(perm_sh.at[j, pl.ds((s % 2) * HALF, HALF)], idxbuf)

    def fire(ck, rbuf_, sem_):
        pltpu.async_copy(
            x_hbm.at[idxbuf.at[pl.ds(ck * CHUNK, CHUNK)]], rbuf_, sem_)

    for bb in range(NBUF):
        fire(bb, rows[bb], sems[bb])

    def g_body(g, carry):
        for bb in range(NBUF):
            ck = g * NBUF + bb
            pltpu.make_async_copy(
                x_hbm.at[idxbuf.at[pl.ds(ck * CHUNK, CHUNK)]], rows[bb],
                sems[bb]).wait()
            pltpu.sync_copy(rows[bb],
                            out_hbm.at[pl.ds(out_base + ck * CHUNK, CHUNK)])

            @pl.when(g < NGRP - 1)
            def _():
                fire(ck + NBUF, rows[bb], sems[bb])
        return carry
    lax.fori_loop(0, NGRP, g_body, jnp.int32(0))


_sorter = pl.kernel(
    _body,
    out_type=jax.ShapeDtypeStruct((B * N, D), jnp.float32),
    mesh=plsc.VectorSubcoreMesh(core_axis_name="c", subcore_axis_name="s"),
    compiler_params=pltpu.CompilerParams(needs_layout_passes=False),
    scratch_types=[
        pltpu.VMEM((N,), jnp.int32),
        pltpu.VMEM((N,), jnp.int32),
        pltpu.VMEM((N,), jnp.int32),
        pltpu.VMEM((N,), jnp.int32),
        pltpu.VMEM((N,), jnp.int32),
        pltpu.VMEM((N,), jnp.int32),
        pltpu.VMEM((4096,), jnp.int32),
        pltpu.VMEM((4096,), jnp.int32),
        pltpu.VMEM((4096,), jnp.int32),
        pltpu.VMEM((N,), jnp.int32),
        pltpu.VMEM((N,), jnp.int32),
        pltpu.SMEM((256,), jnp.int32),
        pltpu.VMEM((HALF,), jnp.int32),
        pltpu.VMEM((CHUNK, D), jnp.float32),
        pltpu.VMEM((CHUNK, D), jnp.float32),
        pltpu.VMEM((CHUNK, D), jnp.float32),
        pltpu.VMEM((CHUNK, D), jnp.float32),
        pltpu.VMEM_SHARED((BPC, N), jnp.int32),
        pltpu.SemaphoreType.DMA,
        pltpu.SemaphoreType.DMA,
        pltpu.SemaphoreType.DMA,
        pltpu.SemaphoreType.DMA,
    ],
)


@jax.jit
def kernel(set_inputs, mag, mask):
    x = set_inputs.reshape(B * N, D)
    mag2 = lax.bitcast_convert_type(mag.reshape(B, N), jnp.int32)
    mask2 = mask.reshape(B, N).astype(jnp.int32)
    out = _sorter(x, mag2, mask2)
    return out.reshape(B, N, D)


# speedup vs baseline: 1.0159x; 1.0159x over previous
import jax
import jax.numpy as jnp
from jax import lax
from jax.experimental import pallas as pl
from jax.experimental.pallas import tpu as pltpu
from jax.experimental.pallas import tpu_sc as plsc

B, N, D = 16, 4096, 256
NC, NS = 2, 16
BPC = B // NC
NV = N // 16
HALF = N // 2
CHUNK = 64
NBUF = 4
NGRP = HALF // (NBUF * CHUNK)
INF_BITS = 0x7F800000


def _body(x_hbm, mag_hbm, mask_hbm, out_hbm,
          fbuf, mbuf, kb0, ib0, kb1, ib1, hist, obuf, dbuf, rnk, tot,
          idxbuf, rows0, rows1, rows2, rows3, perm_sh,
          sem0, sem1, sem2, sem3):
    c = lax.axis_index("c")
    s = lax.axis_index("s")
    lane = lax.iota(jnp.int32, 16)
    rows = [rows0, rows1, rows2, rows3]
    sems = [sem0, sem1, sem2, sem3]

    @pl.when(s < BPC)
    def _sort():
        b = c * BPC + s
        pltpu.sync_copy(mag_hbm.at[b], fbuf)
        pltpu.sync_copy(mask_hbm.at[b], mbuf)

        zeros = jnp.zeros((16,), jnp.int32)
        ones = jnp.ones((16,), jnp.int32)

        @plsc.parallel_loop(0, NV, 1, unroll=4)
        def _init(t):
            ki = jnp.bitwise_and(fbuf[pl.ds(t * 16, 16)], 0x7FFFFFFF)
            m = mbuf[pl.ds(t * 16, 16)]
            e = t * 16 + lane
            a = jnp.bitwise_and(e, 255)
            addr = a * 16 + jnp.bitwise_and(jnp.right_shift(e, 8) + a, 15)
            plsc.store_scatter(kb0, [addr], jnp.where(m != 0, ki, INF_BITS))
            ib0[pl.ds(t * 16, 16)] = jnp.bitwise_and(lane - t, 15) * 256 + t
            hist[pl.ds(t * 16, 16)] = zeros

        bufs = [(kb0, ib0), (kb1, ib1)]
        for p in range(4):
            kin, iin = bufs[p % 2]
            kout, iout = bufs[(p + 1) % 2]
            shift = 8 * p

            @plsc.parallel_loop(0, NV, 1, unroll=4)
            def _dig(t):
                k = kin[pl.ds(t * 16, 16)]
                lv = jnp.bitwise_and(lane - t, 15)
                dbuf[pl.ds(t * 16, 16)] = jnp.bitwise_and(
                    jnp.right_shift(k, shift), 255) * 16 + lv

            def hist_body(t, carry):
                hidx = dbuf[pl.ds(t * 16, 16)]
                r = plsc.load_gather(hist, [hidx])
                rnk[pl.ds(t * 16, 16)] = r
                plsc.addupdate_scatter(hist, [hidx], ones)
                return carry
            lax.fori_loop(0, NV, hist_body, jnp.int32(0), unroll=4)

            @plsc.parallel_loop(0, 256, 1, unroll=4)
            def _scan_a(dd):
                h = hist[pl.ds(dd * 16, 16)]
                obuf[pl.ds(dd * 16, 16)] = jnp.cumsum(h) - h
                tot[dd] = jnp.sum(h)
                hist[pl.ds(dd * 16, 16)] = zeros

            def scan_b(dd, carry):
                t = tot[dd]
                tot[dd] = carry
                return carry + t
            lax.fori_loop(0, 256, scan_b, jnp.int32(0), unroll=4)

            @plsc.parallel_loop(0, 256, 1, unroll=4)
            def _scan_c(dd):
                obuf[pl.ds(dd * 16, 16)] = obuf[pl.ds(dd * 16, 16)] + tot[dd]

            last = p == 3

            @plsc.parallel_loop(0, NV, 1, unroll=4)
            def _perm(t):
                k = kin[pl.ds(t * 16, 16)]
                v = iin[pl.ds(t * 16, 16)]
                hidx = dbuf[pl.ds(t * 16, 16)]
                pos = plsc.load_gather(obuf, [hidx]) + rnk[pl.ds(t * 16, 16)]
                if not last:
                    a2 = jnp.bitwise_and(pos, 255)
                    pos = a2 * 16 + jnp.bitwise_and(
                        jnp.right_shift(pos, 8) + a2, 15)
                plsc.store_scatter(kout, [pos], k)
                plsc.store_scatter(iout, [pos], v)

        base_row = b * N

        @plsc.parallel_loop(0, NV, 1, unroll=4)
        def _add(t):
            ib0[pl.ds(t * 16, 16)] = ib0[pl.ds(t * 16, 16)] + base_row
        pltpu.sync_copy(ib0, perm_sh.at[s])

    plsc.subcore_barrier()

    j = s // 2
    b = c * BPC + j
    out_base = b * N + (s % 2) * HALF
    pltpu.sync_copy(perm_sh.at[j, pl.ds((s % 2) * HALF, HALF)], idxbuf)

    def fire(ck, rbuf_, sem_):
        pltpu.async_copy(
            x_hbm.at[idxbuf.at[pl.ds(ck * CHUNK, CHUNK)]], rbuf_, sem_)

    for bb in range(NBUF):
        fire(bb, rows[bb], sems[bb])

    def g_body(g, carry):
        for bb in range(NBUF):
            ck = g * NBUF + bb
            pltpu.make_async_copy(
                x_hbm.at[idxbuf.at[pl.ds(ck * CHUNK, CHUNK)]], rows[bb],
                sems[bb]).wait()
            pltpu.sync_copy(rows[bb],
                            out_hbm.at[pl.ds(out_base + ck * CHUNK, CHUNK)])

            @pl.when(g < NGRP - 1)
            def _():
                fire(ck + NBUF, rows[bb], sems[bb])
        return carry
    lax.fori_loop(0, NGRP, g_body, jnp.int32(0))


_sorter = pl.kernel(
    _body,
    out_type=jax.ShapeDtypeStruct((B * N, D), jnp.float32),
    mesh=plsc.VectorSubcoreMesh(core_axis_name="c", subcore_axis_name="s"),
    compiler_params=pltpu.CompilerParams(needs_layout_passes=False),
    scratch_types=[
        pltpu.VMEM((N,), jnp.int32),
        pltpu.VMEM((N,), jnp.int32),
        pltpu.VMEM((N,), jnp.int32),
        pltpu.VMEM((N,), jnp.int32),
        pltpu.VMEM((N,), jnp.int32),
        pltpu.VMEM((N,), jnp.int32),
        pltpu.VMEM((4096,), jnp.int32),
        pltpu.VMEM((4096,), jnp.int32),
        pltpu.VMEM((N,), jnp.int32),
        pltpu.VMEM((N,), jnp.int32),
        pltpu.SMEM((256,), jnp.int32),
        pltpu.VMEM((HALF,), jnp.int32),
        pltpu.VMEM((CHUNK, D), jnp.float32),
        pltpu.VMEM((CHUNK, D), jnp.float32),
        pltpu.VMEM((CHUNK, D), jnp.float32),
        pltpu.VMEM((CHUNK, D), jnp.float32),
        pltpu.VMEM_SHARED((BPC, N), jnp.int32),
        pltpu.SemaphoreType.DMA,
        pltpu.SemaphoreType.DMA,
        pltpu.SemaphoreType.DMA,
        pltpu.SemaphoreType.DMA,
    ],
)


@jax.jit
def kernel(set_inputs, mag, mask):
    x = set_inputs.reshape(B * N, D)
    mag2 = lax.bitcast_convert_type(mag.reshape(B, N), jnp.int32)
    mask2 = mask.reshape(B, N).astype(jnp.int32)
    out = _sorter(x, mag2, mask2)
    return out.reshape(B, N, D)
